# baseline (speedup 1.0000x reference)
import functools

import jax
import jax.numpy as jnp
from jax import lax
from jax.experimental import pallas as pl
from jax.experimental.pallas import tpu as pltpu
from jax.experimental.pallas import tpu_sc as plsc

NC = 2
NS = 16
CH = 128
G = 4


def _sc_aggregate(h2, e):
    _, n_pad, dh = h2.shape
    ngrp = e.shape[2]
    rpt = n_pad // NS

    mesh = plsc.VectorSubcoreMesh(core_axis_name="c", subcore_axis_name="s")

    @functools.partial(
        pl.kernel,
        out_type=jax.ShapeDtypeStruct((NC, n_pad, dh), jnp.float32),
        mesh=mesh,
        scratch_types=[
            pltpu.VMEM((2, 2, G, CH), jnp.int32),
            pltpu.VMEM((2, CH, dh), jnp.float32),
            pltpu.VMEM_SHARED((n_pad, dh), jnp.float32),
            pltpu.VMEM_SHARED((n_pad, dh), jnp.float32),
            pltpu.SemaphoreType.DMA((2,)),
            pltpu.SemaphoreType.DMA((2,)),
        ],
        compiler_params=pltpu.CompilerParams(use_tc_tiling_on_sc=False),
    )
    def sc_k(h2_hbm, e_hbm, out_hbm, idx_v, rows_v, tab_sh, acc_sh,
             gsem, isem):
        c = lax.axis_index("c")
        s = lax.axis_index("s")
        r0 = s * rpt
        last = ngrp - 1

        def idx_dma(g, ib):
            return (pltpu.make_async_copy(e_hbm.at[0, s, g], idx_v.at[ib, 0],
                                          isem.at[ib]),
                    pltpu.make_async_copy(e_hbm.at[1, s, g], idx_v.at[ib, 1],
                                          isem.at[ib]))

        def idx_start(g, ib):
            for d_ in idx_dma(g, ib):
                d_.start()

        def idx_wait(ib):
            for d_ in idx_dma(0, ib):
                d_.wait()

        def gather(ib, j, rb):
            return pltpu.make_async_copy(tab_sh.at[idx_v.at[ib, 0, j]],
                                         rows_v.at[rb], gsem.at[rb])

        idx_start(0, 0)
        idx_start(jnp.minimum(1, last), 1)
        zv = jnp.zeros((16,), jnp.float32)

        @pl.loop(0, CH)
        def _(i):
            for rb in range(2):
                for j in range(dh // 16):
                    rows_v[rb, i, pl.ds(j * 16, 16)] = zv

        off = 0
        while off < rpt:
            m = min(rpt - off, CH)
            pltpu.sync_copy(rows_v.at[0, pl.ds(0, m)],
                            acc_sh.at[pl.ds(r0 + off, m)])
            off += m
        pltpu.sync_copy(h2_hbm.at[c, pl.ds(r0, rpt)], tab_sh.at[pl.ds(r0, rpt)])
        plsc.subcore_barrier()

        idx_wait(0)
        gather(0, 0, 0).start()
        gather(0, 1, 1).start()

        @pl.loop(0, ngrp, step=2)
        def _(g):
            for gb in range(2):
                cg = g + gb
                for j in range(G):
                    rb = j % 2
                    gather(gb, j, rb).wait()
                    pltpu.sync_copy(rows_v.at[rb],
                                    acc_sh.at[idx_v.at[gb, 1, j]], add=True)
                    if j == G - 2:
                        idx_wait(gb ^ 1)
                    if j < G - 2:
                        gather(gb, j + 2, rb).start()
                    else:
                        gather(gb ^ 1, j + 2 - G, rb).start()
                    if j == G - 1:
                        idx_start(jnp.minimum(cg + 2, last), gb)

        idx_wait(1)
        gather(0, 0, 0).wait()
        gather(0, 1, 1).wait()

        plsc.subcore_barrier()
        pltpu.sync_copy(acc_sh.at[pl.ds(r0, rpt)], out_hbm.at[c, pl.ds(r0, rpt)])

    return sc_k(h2, e)


def _tc_finish(acc, h, w_self, wn2, b2, cnt_col):
    n, d = h.shape
    dh = acc.shape[2]
    d_out = w_self.shape[1]
    blk = 1000 if n % 1000 == 0 else 8
    grid = n // blk

    def body(acc_ref, h_ref, ws_ref, wn_ref, b_ref, o_ref):
        p0 = acc_ref[0]
        p1 = acc_ref[1]
        cnt = p1[:, cnt_col:cnt_col + 1]
        neigh = (
            jnp.dot(p0, wn_ref[0], preferred_element_type=jnp.float32)
            + jnp.dot(p1, wn_ref[1], preferred_element_type=jnp.float32)
        ) / jnp.maximum(cnt, 1.0)
        o_ref[...] = (
            jnp.dot(h_ref[...], ws_ref[...], preferred_element_type=jnp.float32)
            + neigh + b_ref[...]
        )

    return pl.pallas_call(
        body,
        grid=(grid,),
        in_specs=[
            pl.BlockSpec((2, blk, dh), lambda i: (0, i, 0)),
            pl.BlockSpec((blk, d), lambda i: (i, 0)),
            pl.BlockSpec((d, d_out), lambda i: (0, 0)),
            pl.BlockSpec((2, dh, d_out), lambda i: (0, 0, 0)),
            pl.BlockSpec((1, d_out), lambda i: (0, 0)),
        ],
        out_specs=pl.BlockSpec((blk, d_out), lambda i: (i, 0)),
        out_shape=jax.ShapeDtypeStruct((n, d_out), jnp.float32),
    )(acc, h, w_self, wn2, b2)


def kernel(h, edge_index, W, b):
    n, d = h.shape
    e_cnt = edge_index.shape[1]
    da = ((d + 1 + 31) // 32) * 32
    dh = da // 2
    n_pad = ((n + 1 + NS * 8 - 1) // (NS * 8)) * (NS * 8)
    step = NS * CH * G * 2
    e_pad_cnt = ((e_cnt + step - 1) // step) * step
    ngrp = e_pad_cnt // (NS * CH * G)

    e32 = edge_index.astype(jnp.int32)
    if e_pad_cnt != e_cnt:
        npe = e_pad_cnt - e_cnt
        pad_dst = n + jnp.arange(npe, dtype=jnp.int32) % (n_pad - n)
        pad = jnp.stack([jnp.full((npe,), n, jnp.int32), pad_dst])
        e32 = jnp.concatenate([e32, pad], axis=1)
    e32 = e32.reshape(2, NS, ngrp, G, CH)

    top = jnp.zeros((n_pad, dh), jnp.float32).at[:n].set(h[:, :dh])
    bot = jnp.zeros((n_pad, dh), jnp.float32)
    bot = bot.at[:n, :d - dh].set(h[:, dh:]).at[:n, d - dh].set(1.0)
    h2 = jnp.stack([top, bot])

    acc = _sc_aggregate(h2, e32)

    wn = W[d:]
    wn2 = jnp.zeros((2, dh, W.shape[1]), jnp.float32)
    wn2 = wn2.at[0].set(wn[:dh]).at[1, :d - dh].set(wn[dh:])
    cnt_col = d - dh
    return _tc_finish(acc, h, W[:d], wn2, b.reshape(1, -1), cnt_col)


# speedup vs baseline: 1.0062x; 1.0062x over previous
import functools

import jax
import jax.numpy as jnp
from jax import lax
from jax.experimental import pallas as pl
from jax.experimental.pallas import tpu as pltpu
from jax.experimental.pallas import tpu_sc as plsc

NC = 2
NS = 16
CH = 128
G = 3


def _sc_aggregate(h2, e, n, n_pad):
    _, _, dh = h2.shape
    e_cnt = e.shape[1]
    ept = e_cnt // NS
    full = ept // CH
    ngrp = full // G
    extra = list(range(ngrp * G, full))
    tail = ept - full * CH
    assert ngrp % 2 == 0 and tail % 8 == 0
    rpt = n_pad // NS

    mesh = plsc.VectorSubcoreMesh(core_axis_name="c", subcore_axis_name="s")

    @functools.partial(
        pl.kernel,
        out_type=jax.ShapeDtypeStruct((NC, n_pad, dh), jnp.float32),
        mesh=mesh,
        scratch_types=[
            pltpu.VMEM((2, 2, G, CH), jnp.int32),
            pltpu.VMEM((2, CH, dh), jnp.float32),
            pltpu.VMEM_SHARED((n_pad, dh), jnp.float32),
            pltpu.VMEM_SHARED((n_pad, dh), jnp.float32),
            pltpu.SemaphoreType.DMA((2,)),
            pltpu.SemaphoreType.DMA((2,)),
        ],
        compiler_params=pltpu.CompilerParams(use_tc_tiling_on_sc=False),
    )
    def sc_k(h2_hbm, e_hbm, out_hbm, idx_v, rows_v, tab_sh, acc_sh,
             gsem, isem):
        c = lax.axis_index("c")
        s = lax.axis_index("s")
        r0 = s * rpt
        base = s * ept
        last = ngrp - 1

        def idx_dma(g, ib):
            ds = []
            for j in range(G):
                off = base + (g * G + j) * CH
                for sd in range(2):
                    ds.append(pltpu.make_async_copy(
                        e_hbm.at[sd, pl.ds(off, CH)], idx_v.at[ib, sd, j],
                        isem.at[ib]))
            return ds

        def idx_start(g, ib):
            for d_ in idx_dma(g, ib):
                d_.start()

        def idx_wait(ib):
            for d_ in idx_dma(0, ib):
                d_.wait()

        def gather(ib, j, rb):
            return pltpu.make_async_copy(tab_sh.at[idx_v.at[ib, 0, j]],
                                         rows_v.at[rb], gsem.at[rb])

        idx_start(0, 0)
        idx_start(jnp.minimum(1, last), 1)
        zv = jnp.zeros((16,), jnp.float32)

        @pl.loop(0, CH)
        def _(i):
            for rb in range(2):
                for j in range(dh // 16):
                    rows_v[rb, i, pl.ds(j * 16, 16)] = zv

        off = 0
        while off < rpt:
            m = min(rpt - off, CH)
            pltpu.sync_copy(rows_v.at[0, pl.ds(0, m)],
                            acc_sh.at[pl.ds(r0 + off, m)])
            off += m
        pltpu.sync_copy(h2_hbm.at[c, pl.ds(r0, rpt)], tab_sh.at[pl.ds(r0, rpt)])
        plsc.subcore_barrier()

        idx_wait(0)
        gather(0, 0, 0).start()
        gather(0, 1, 1).start()

        @pl.loop(0, ngrp, step=2)
        def _(g):
            for gb in range(2):
                cg = g + gb
                for j in range(G):
                    rb = j % 2
                    gather(gb, j, rb).wait()
                    pltpu.sync_copy(rows_v.at[rb],
                                    acc_sh.at[idx_v.at[gb, 1, j]], add=True)
                    if j == G - 2:
                        idx_wait(gb ^ 1)
                    if j < G - 2:
                        gather(gb, j + 2, rb).start()
                    else:
                        gather(gb ^ 1, j + 2 - G, rb).start()
                    if j == G - 1:
                        idx_start(jnp.minimum(cg + 2, last), gb)

        idx_wait(1)
        gather(0, 0, 0).wait()
        gather(0, 1, 1).wait()

        for t in extra:
            off = base + t * CH
            pltpu.sync_copy(e_hbm.at[0, pl.ds(off, CH)], idx_v.at[0, 0, 0])
            pltpu.sync_copy(e_hbm.at[1, pl.ds(off, CH)], idx_v.at[0, 1, 0])
            pltpu.sync_copy(tab_sh.at[idx_v.at[0, 0, 0]], rows_v.at[0])
            pltpu.sync_copy(rows_v.at[0], acc_sh.at[idx_v.at[0, 1, 0]],
                            add=True)
        if tail:
            toff = base + full * CH
            pltpu.sync_copy(e_hbm.at[0, pl.ds(toff, tail)],
                            idx_v.at[0, 0, 0, pl.ds(0, tail)])
            pltpu.sync_copy(e_hbm.at[1, pl.ds(toff, tail)],
                            idx_v.at[0, 1, 0, pl.ds(0, tail)])
            lane = lax.iota(jnp.int32, 16)
            for q in range(tail // 16, CH // 16):
                idx_v[0, 1, 0, pl.ds(q * 16, 16)] = (
                    n + (q * 16 + lane) % (n_pad - n))
            pltpu.sync_copy(tab_sh.at[idx_v.at[0, 0, 0]], rows_v.at[0])
            pltpu.sync_copy(rows_v.at[0], acc_sh.at[idx_v.at[0, 1, 0]],
                            add=True)

        plsc.subcore_barrier()
        pltpu.sync_copy(acc_sh.at[pl.ds(r0, rpt)], out_hbm.at[c, pl.ds(r0, rpt)])

    return sc_k(h2, e)


def _tc_finish(acc, h, w_self, wn2, b2, cnt_col):
    n, d = h.shape
    dh = acc.shape[2]
    d_out = w_self.shape[1]
    blk = 1000 if n % 1000 == 0 else 8
    grid = n // blk

    def body(acc_ref, h_ref, ws_ref, wn_ref, b_ref, o_ref):
        p0 = acc_ref[0]
        p1 = acc_ref[1]
        cnt = p1[:, cnt_col:cnt_col + 1]
        neigh = (
            jnp.dot(p0, wn_ref[0], preferred_element_type=jnp.float32)
            + jnp.dot(p1, wn_ref[1], preferred_element_type=jnp.float32)
        ) / jnp.maximum(cnt, 1.0)
        o_ref[...] = (
            jnp.dot(h_ref[...], ws_ref[...], preferred_element_type=jnp.float32)
            + neigh + b_ref[...]
        )

    return pl.pallas_call(
        body,
        grid=(grid,),
        in_specs=[
            pl.BlockSpec((2, blk, dh), lambda i: (0, i, 0)),
            pl.BlockSpec((blk, d), lambda i: (i, 0)),
            pl.BlockSpec((d, d_out), lambda i: (0, 0)),
            pl.BlockSpec((2, dh, d_out), lambda i: (0, 0, 0)),
            pl.BlockSpec((1, d_out), lambda i: (0, 0)),
        ],
        out_specs=pl.BlockSpec((blk, d_out), lambda i: (i, 0)),
        out_shape=jax.ShapeDtypeStruct((n, d_out), jnp.float32),
    )(acc, h, w_self, wn2, b2)


def kernel(h, edge_index, W, b):
    n, d = h.shape
    e_cnt = edge_index.shape[1]
    da = ((d + 1 + 31) // 32) * 32
    dh = da // 2
    n_pad = ((n + 1 + NS * 8 - 1) // (NS * 8)) * (NS * 8)

    del e_cnt
    e32 = edge_index.astype(jnp.int32)

    top = jnp.pad(h[:, :dh], ((0, n_pad - n), (0, 0)))
    bot = jnp.pad(
        jnp.concatenate([h[:, dh:], jnp.ones((n, 1), jnp.float32)], axis=1),
        ((0, n_pad - n), (0, dh - (d - dh) - 1)))
    h2 = jnp.stack([top, bot])

    acc = _sc_aggregate(h2, e32, n, n_pad)

    wn = W[d:]
    wn2 = jnp.zeros((2, dh, W.shape[1]), jnp.float32)
    wn2 = wn2.at[0].set(wn[:dh]).at[1, :d - dh].set(wn[dh:])
    cnt_col = d - dh
    return _tc_finish(acc, h, W[:d], wn2, b.reshape(1, -1), cnt_col)


# speedup vs baseline: 1.1620x; 1.1549x over previous
import functools

import jax
import jax.numpy as jnp
from jax import lax
from jax.experimental import pallas as pl
from jax.experimental.pallas import tpu as pltpu
from jax.experimental.pallas import tpu_sc as plsc

NC = 2
NS = 16
CH = 128
G = 6


def _sc_aggregate(h2, e, n, n_pad):
    _, _, dh = h2.shape
    e_cnt = e.shape[1]
    ept = e_cnt // NS
    full = ept // CH
    ngrp = full // G
    extra = list(range(ngrp * G, full))
    tail = ept - full * CH
    assert ngrp % 2 == 0 and tail % 8 == 0
    rpt = n_pad // NS

    mesh = plsc.VectorSubcoreMesh(core_axis_name="c", subcore_axis_name="s")

    @functools.partial(
        pl.kernel,
        out_type=jax.ShapeDtypeStruct((NC, n_pad, dh), jnp.float32),
        mesh=mesh,
        scratch_types=[
            pltpu.VMEM((2, 2, G, CH), jnp.int32),
            pltpu.VMEM((2, CH, dh), jnp.float32),
            pltpu.VMEM_SHARED((n_pad, dh), jnp.float32),
            pltpu.VMEM_SHARED((n_pad, dh), jnp.float32),
            pltpu.SemaphoreType.DMA((2,)),
            pltpu.SemaphoreType.DMA((2,)),
        ],
        compiler_params=pltpu.CompilerParams(use_tc_tiling_on_sc=False),
    )
    def sc_k(h2_hbm, e_hbm, out_hbm, idx_v, rows_v, tab_sh, acc_sh,
             gsem, isem):
        c = lax.axis_index("c")
        s = lax.axis_index("s")
        r0 = s * rpt
        base = s * ept
        last = ngrp - 1

        def idx_dma(g, ib):
            ds = []
            for j in range(G):
                off = base + (g * G + j) * CH
                for sd in range(2):
                    ds.append(pltpu.make_async_copy(
                        e_hbm.at[sd, pl.ds(off, CH)], idx_v.at[ib, sd, j],
                        isem.at[ib]))
            return ds

        def idx_start(g, ib):
            for d_ in idx_dma(g, ib):
                d_.start()

        def idx_wait(ib):
            for d_ in idx_dma(0, ib):
                d_.wait()

        def gather(ib, j, rb):
            return pltpu.make_async_copy(tab_sh.at[idx_v.at[ib, 0, j]],
                                         rows_v.at[rb], gsem.at[rb])

        idx_start(0, 0)
        idx_start(jnp.minimum(1, last), 1)
        zv = jnp.zeros((16,), jnp.float32)

        @pl.loop(0, CH)
        def _(i):
            for rb in range(2):
                for j in range(dh // 16):
                    rows_v[rb, i, pl.ds(j * 16, 16)] = zv

        off = 0
        while off < rpt:
            m = min(rpt - off, CH)
            pltpu.sync_copy(rows_v.at[0, pl.ds(0, m)],
                            acc_sh.at[pl.ds(r0 + off, m)])
            off += m
        pltpu.sync_copy(h2_hbm.at[c, pl.ds(r0, rpt)], tab_sh.at[pl.ds(r0, rpt)])
        plsc.subcore_barrier()

        idx_wait(0)
        gather(0, 0, 0).start()
        gather(0, 1, 1).start()

        @pl.loop(0, ngrp, step=2)
        def _(g):
            for gb in range(2):
                cg = g + gb
                for j in range(G):
                    rb = j % 2
                    gather(gb, j, rb).wait()
                    pltpu.sync_copy(rows_v.at[rb],
                                    acc_sh.at[idx_v.at[gb, 1, j]], add=True)
                    if j == G - 2:
                        idx_wait(gb ^ 1)
                    if j < G - 2:
                        gather(gb, j + 2, rb).start()
                    else:
                        gather(gb ^ 1, j + 2 - G, rb).start()
                    if j == G - 1:
                        idx_start(jnp.minimum(cg + 2, last), gb)

        idx_wait(1)
        gather(0, 0, 0).wait()
        gather(0, 1, 1).wait()

        for t in extra:
            off = base + t * CH
            pltpu.sync_copy(e_hbm.at[0, pl.ds(off, CH)], idx_v.at[0, 0, 0])
            pltpu.sync_copy(e_hbm.at[1, pl.ds(off, CH)], idx_v.at[0, 1, 0])
            pltpu.sync_copy(tab_sh.at[idx_v.at[0, 0, 0]], rows_v.at[0])
            pltpu.sync_copy(rows_v.at[0], acc_sh.at[idx_v.at[0, 1, 0]],
                            add=True)
        if tail:
            toff = base + full * CH
            pltpu.sync_copy(e_hbm.at[0, pl.ds(toff, tail)],
                            idx_v.at[0, 0, 0, pl.ds(0, tail)])
            pltpu.sync_copy(e_hbm.at[1, pl.ds(toff, tail)],
                            idx_v.at[0, 1, 0, pl.ds(0, tail)])
            lane = lax.iota(jnp.int32, 16)
            for q in range(tail // 16, CH // 16):
                idx_v[0, 1, 0, pl.ds(q * 16, 16)] = (
                    n + (q * 16 + lane) % (n_pad - n))
            pltpu.sync_copy(tab_sh.at[idx_v.at[0, 0, 0]], rows_v.at[0])
            pltpu.sync_copy(rows_v.at[0], acc_sh.at[idx_v.at[0, 1, 0]],
                            add=True)

        plsc.subcore_barrier()
        pltpu.sync_copy(acc_sh.at[pl.ds(r0, rpt)], out_hbm.at[c, pl.ds(r0, rpt)])

    return sc_k(h2, e)


def _tc_finish(acc, h, w_self, wn2, b2, cnt_col):
    n, d = h.shape
    dh = acc.shape[2]
    d_out = w_self.shape[1]
    blk = 1000 if n % 1000 == 0 else 8
    grid = n // blk

    def body(acc_ref, h_ref, ws_ref, wn_ref, b_ref, o_ref):
        p0 = acc_ref[0]
        p1 = acc_ref[1]
        cnt = p1[:, cnt_col:cnt_col + 1]
        neigh = (
            jnp.dot(p0, wn_ref[0], preferred_element_type=jnp.float32)
            + jnp.dot(p1, wn_ref[1], preferred_element_type=jnp.float32)
        ) / jnp.maximum(cnt, 1.0)
        o_ref[...] = (
            jnp.dot(h_ref[...], ws_ref[...], preferred_element_type=jnp.float32)
            + neigh + b_ref[...]
        )

    return pl.pallas_call(
        body,
        grid=(grid,),
        in_specs=[
            pl.BlockSpec((2, blk, dh), lambda i: (0, i, 0)),
            pl.BlockSpec((blk, d), lambda i: (i, 0)),
            pl.BlockSpec((d, d_out), lambda i: (0, 0)),
            pl.BlockSpec((2, dh, d_out), lambda i: (0, 0, 0)),
            pl.BlockSpec((1, d_out), lambda i: (0, 0)),
        ],
        out_specs=pl.BlockSpec((blk, d_out), lambda i: (i, 0)),
        out_shape=jax.ShapeDtypeStruct((n, d_out), jnp.float32),
    )(acc, h, w_self, wn2, b2)


def kernel(h, edge_index, W, b):
    n, d = h.shape
    e_cnt = edge_index.shape[1]
    da = ((d + 1 + 31) // 32) * 32
    dh = da // 2
    n_pad = ((n + 1 + NS * 8 - 1) // (NS * 8)) * (NS * 8)

    del e_cnt
    e32 = edge_index.astype(jnp.int32)

    top = jnp.pad(h[:, :dh], ((0, n_pad - n), (0, 0)))
    bot = jnp.pad(
        jnp.concatenate([h[:, dh:], jnp.ones((n, 1), jnp.float32)], axis=1),
        ((0, n_pad - n), (0, dh - (d - dh) - 1)))
    h2 = jnp.stack([top, bot])

    acc = _sc_aggregate(h2, e32, n, n_pad)

    wn = W[d:]
    wn2 = jnp.zeros((2, dh, W.shape[1]), jnp.float32)
    wn2 = wn2.at[0].set(wn[:dh]).at[1, :d - dh].set(wn[dh:])
    cnt_col = d - dh
    return _tc_finish(acc, h, W[:d], wn2, b.reshape(1, -1), cnt_col)


# speedup vs baseline: 1.4068x; 1.2107x over previous
import functools

import jax
import jax.numpy as jnp
from jax import lax
from jax.experimental import pallas as pl
from jax.experimental.pallas import tpu as pltpu
from jax.experimental.pallas import tpu_sc as plsc

NC = 2
NS = 16
CH = 128
G = 6


def _sc_aggregate(h2, e, n, n_pad):
    _, _, dh = h2.shape
    e_cnt = e.shape[1]
    ept = e_cnt // NS
    full = ept // CH
    ngrp = full // G
    extra = list(range(ngrp * G, full))
    tail = ept - full * CH
    assert ngrp % 2 == 0 and tail % 8 == 0
    rpt = n_pad // NS

    mesh = plsc.VectorSubcoreMesh(core_axis_name="c", subcore_axis_name="s")

    @functools.partial(
        pl.kernel,
        out_type=jax.ShapeDtypeStruct((NC, n_pad, dh), jnp.float32),
        mesh=mesh,
        scratch_types=[
            pltpu.VMEM((2, 2, G, CH), jnp.int32),
            pltpu.VMEM((4, CH, dh), jnp.float32),
            pltpu.VMEM_SHARED((n_pad, dh), jnp.float32),
            pltpu.SemaphoreType.DMA((4,)),
            pltpu.SemaphoreType.DMA((2,)),
            pltpu.SemaphoreType.DMA((4,)),
        ],
        compiler_params=pltpu.CompilerParams(use_tc_tiling_on_sc=False),
    )
    def sc_k(h2_hbm, e_hbm, out_hbm, idx_v, rows_v, acc_sh,
             gsem, isem, ssem):
        c = lax.axis_index("c")
        s = lax.axis_index("s")
        r0 = s * rpt
        base = s * ept
        last = ngrp - 1

        def idx_dma(g, ib):
            ds = []
            for j in range(G):
                off = base + (g * G + j) * CH
                for sd in range(2):
                    ds.append(pltpu.make_async_copy(
                        e_hbm.at[sd, pl.ds(off, CH)], idx_v.at[ib, sd, j],
                        isem.at[ib]))
            return ds

        def idx_start(g, ib):
            for d_ in idx_dma(g, ib):
                d_.start()

        def idx_wait(ib):
            for d_ in idx_dma(0, ib):
                d_.wait()

        def gather(ib, j, rb):
            return pltpu.make_async_copy(h2_hbm.at[c].at[idx_v.at[ib, 0, j]],
                                         rows_v.at[rb], gsem.at[rb])

        def scat(ib, j, rb):
            return pltpu.make_async_copy(rows_v.at[rb],
                                         acc_sh.at[idx_v.at[ib, 1, j]],
                                         ssem.at[rb])

        idx_start(0, 0)
        idx_start(jnp.minimum(1, last), 1)
        zv = jnp.zeros((16,), jnp.float32)

        @pl.loop(0, CH)
        def _(i):
            for rb in range(2):
                for j in range(dh // 16):
                    rows_v[rb, i, pl.ds(j * 16, 16)] = zv

        off = 0
        while off < rpt:
            m = min(rpt - off, CH)
            pltpu.sync_copy(rows_v.at[0, pl.ds(0, m)],
                            acc_sh.at[pl.ds(r0 + off, m)])
            off += m
        plsc.subcore_barrier()

        idx_wait(0)
        gather(0, 0, 0).start()
        gather(0, 1, 1).start()

        def dgrp(g, first):
            for gb in range(2):
                for j in range(G):
                    cl = gb * G + j
                    b = cl % 4
                    b2 = (cl + 2) % 4
                    gather(gb, j, b).wait()
                    scat(gb, j, b).start(add=True)
                    if j == G - 2:
                        idx_wait(gb ^ 1)
                    if not (first and cl < 2):
                        scat(0, 0, b2).wait()
                    if j < G - 2:
                        gather(gb, j + 2, b2).start()
                    else:
                        gather(gb ^ 1, j + 2 - G, b2).start()
                    if j == G - 1:
                        idx_start(jnp.minimum(g + gb + 2, last), gb)

        dgrp(0, True)

        @pl.loop(2, ngrp, step=2)
        def _(g):
            dgrp(g, False)

        total = ngrp * G
        scat(0, 0, (total - 2) % 4).wait()
        scat(0, 0, (total - 1) % 4).wait()
        idx_wait(1)
        gather(0, 0, total % 4).wait()
        gather(0, 1, (total + 1) % 4).wait()

        for t in extra:
            off = base + t * CH
            pltpu.sync_copy(e_hbm.at[0, pl.ds(off, CH)], idx_v.at[0, 0, 0])
            pltpu.sync_copy(e_hbm.at[1, pl.ds(off, CH)], idx_v.at[0, 1, 0])
            pltpu.sync_copy(h2_hbm.at[c].at[idx_v.at[0, 0, 0]], rows_v.at[0])
            pltpu.sync_copy(rows_v.at[0], acc_sh.at[idx_v.at[0, 1, 0]],
                            add=True)
        if tail:
            toff = base + full * CH
            pltpu.sync_copy(e_hbm.at[0, pl.ds(toff, tail)],
                            idx_v.at[0, 0, 0, pl.ds(0, tail)])
            pltpu.sync_copy(e_hbm.at[1, pl.ds(toff, tail)],
                            idx_v.at[0, 1, 0, pl.ds(0, tail)])
            lane = lax.iota(jnp.int32, 16)
            for q in range(tail // 16, CH // 16):
                idx_v[0, 1, 0, pl.ds(q * 16, 16)] = (
                    n + (q * 16 + lane) % (n_pad - n))
            pltpu.sync_copy(h2_hbm.at[c].at[idx_v.at[0, 0, 0]], rows_v.at[0])
            pltpu.sync_copy(rows_v.at[0], acc_sh.at[idx_v.at[0, 1, 0]],
                            add=True)

        plsc.subcore_barrier()
        pltpu.sync_copy(acc_sh.at[pl.ds(r0, rpt)], out_hbm.at[c, pl.ds(r0, rpt)])

    return sc_k(h2, e)


def _tc_finish(acc, h, w_self, wn2, b2, cnt_col):
    n, d = h.shape
    dh = acc.shape[2]
    d_out = w_self.shape[1]
    blk = 1000 if n % 1000 == 0 else 8
    grid = n // blk

    def body(acc_ref, h_ref, ws_ref, wn_ref, b_ref, o_ref):
        p0 = acc_ref[0]
        p1 = acc_ref[1]
        cnt = p1[:, cnt_col:cnt_col + 1]
        neigh = (
            jnp.dot(p0, wn_ref[0], preferred_element_type=jnp.float32)
            + jnp.dot(p1, wn_ref[1], preferred_element_type=jnp.float32)
        ) / jnp.maximum(cnt, 1.0)
        o_ref[...] = (
            jnp.dot(h_ref[...], ws_ref[...], preferred_element_type=jnp.float32)
            + neigh + b_ref[...]
        )

    return pl.pallas_call(
        body,
        grid=(grid,),
        in_specs=[
            pl.BlockSpec((2, blk, dh), lambda i: (0, i, 0)),
            pl.BlockSpec((blk, d), lambda i: (i, 0)),
            pl.BlockSpec((d, d_out), lambda i: (0, 0)),
            pl.BlockSpec((2, dh, d_out), lambda i: (0, 0, 0)),
            pl.BlockSpec((1, d_out), lambda i: (0, 0)),
        ],
        out_specs=pl.BlockSpec((blk, d_out), lambda i: (i, 0)),
        out_shape=jax.ShapeDtypeStruct((n, d_out), jnp.float32),
    )(acc, h, w_self, wn2, b2)


def kernel(h, edge_index, W, b):
    n, d = h.shape
    e_cnt = edge_index.shape[1]
    da = ((d + 1 + 31) // 32) * 32
    dh = da // 2
    n_pad = ((n + 1 + NS * 8 - 1) // (NS * 8)) * (NS * 8)

    del e_cnt
    e32 = edge_index.astype(jnp.int32)

    top = jnp.pad(h[:, :dh], ((0, n_pad - n), (0, 0)))
    bot = jnp.pad(
        jnp.concatenate([h[:, dh:], jnp.ones((n, 1), jnp.float32)], axis=1),
        ((0, n_pad - n), (0, dh - (d - dh) - 1)))
    h2 = jnp.stack([top, bot])

    acc = _sc_aggregate(h2, e32, n, n_pad)

    wn = W[d:]
    wn2 = jnp.zeros((2, dh, W.shape[1]), jnp.float32)
    wn2 = wn2.at[0].set(wn[:dh]).at[1, :d - dh].set(wn[dh:])
    cnt_col = d - dh
    return _tc_finish(acc, h, W[:d], wn2, b.reshape(1, -1), cnt_col)
